# Initial kernel scaffold; baseline (speedup 1.0000x reference)
import jax
import jax.numpy as jnp
from jax.experimental import pallas as pl


def kernel(f1_lookup, f2_lookup, f3_content, f3_lookup, f4_content, f4_lookup, f5_lookup, f6_time, f6_lookup, f6_week, f6_day, emb1, emb2, W3, b3, W4, b4, emb5, W6, b6, week_tab, day_tab):
    raise NotImplementedError("write your pallas kernel here")


# speedup vs baseline: 1.6447x; 1.6447x over previous
import functools
import math

import jax
import jax.numpy as jnp
from jax import lax
from jax.experimental import pallas as pl
from jax.experimental.pallas import tpu as pltpu
from jax.experimental.pallas import tpu_sc as plsc

N, L = 1024, 50
B = N * L
D = 128
N_BANDS = 8

_NC, _NS = 2, 16
_NW = _NC * _NS
_BPW = B // _NW
_HALF = 800
_CH = 80
_NFIRE = _HALF // _CH


def _fourier_w(n_bands, offset):
    steps = n_bands + offset + 1
    w = 2.0 ** jnp.linspace(-float(n_bands), float(offset), steps)
    return (w * math.pi).astype(jnp.float32)


def _make_sc_gather():
    mesh = plsc.VectorSubcoreMesh(core_axis_name="c", subcore_axis_name="s")

    @functools.partial(
        pl.kernel,
        mesh=mesh,
        out_type=(
            jax.ShapeDtypeStruct((B, D), jnp.float32),
            jax.ShapeDtypeStruct((B, D), jnp.float32),
            jax.ShapeDtypeStruct((B, D), jnp.float32),
        ),
        scratch_types=[
            pltpu.VMEM((_HALF,), jnp.int32),
            pltpu.VMEM((_HALF, D), jnp.float32),
            pltpu.SemaphoreType.DMA,
        ],
    )
    def sc_gather(i1, i2, i5, t1, t2, t5, o1, o2, o5, idx_v, rows_v, sem):
        wid = lax.axis_index("s") * _NC + lax.axis_index("c")
        base0 = wid * _BPW

        for ih, th, oh in ((i1, t1, o1), (i2, t2, o2), (i5, t5, o5)):
            def round_body(r, carry, ih=ih, th=th, oh=oh):
                base = base0 + r * _HALF
                pltpu.sync_copy(ih.at[pl.ds(base, _HALF)], idx_v)
                for j in range(_NFIRE):
                    pltpu.async_copy(
                        th.at[idx_v.at[pl.ds(j * _CH, _CH)]],
                        rows_v.at[pl.ds(j * _CH, _CH)],
                        sem,
                    )
                for j in range(_NFIRE):
                    pltpu.make_async_copy(
                        th.at[idx_v.at[pl.ds(j * _CH, _CH)]],
                        rows_v.at[pl.ds(j * _CH, _CH)],
                        sem,
                    ).wait()
                pltpu.sync_copy(rows_v, oh.at[pl.ds(base, _HALF)])
                return carry

            lax.fori_loop(0, _BPW // _HALF, round_body, 0)

    return sc_gather


_T = 1024
_GRID = B // _T


def _tc_body(x3, l3, x4, l4, x6, l6, wk, dy, g1, g2, g5,
             wcp, wtp, w3s, w3c, w4s, w4c, w6s, w6c, b3, b4, b6, cdt,
             out_ref):
    f32 = jnp.float32

    def cont(x_ref, l_ref, wp_ref, ws_ref, wc_ref, b_ref):
        a = (x_ref[...] - l_ref[...]) * wp_ref[...]
        e = jnp.dot(jnp.sin(a), ws_ref[...], preferred_element_type=f32)
        e += jnp.dot(jnp.cos(a), wc_ref[...], preferred_element_type=f32)
        return e + b_ref[...]

    e3 = cont(x3, l3, wcp, w3s, w3c, b3)
    e4 = cont(x4, l4, wcp, w4s, w4c, b4)
    e6 = cont(x6, l6, wtp, w6s, w6c, b6)

    lanes = lax.broadcasted_iota(jnp.int32, (_T, 128), 1)
    oh = (lanes == wk[...]).astype(f32) + (lanes == dy[...] + 64).astype(f32)
    e6 += jnp.dot(oh, cdt[...], preferred_element_type=f32)

    out_ref[:, 0, :] = g1[...]
    out_ref[:, 1, :] = g2[...]
    out_ref[:, 2, :] = e3
    out_ref[:, 3, :] = e4
    out_ref[:, 4, :] = g5[...]
    out_ref[:, 5, :] = e6


def _pad_rows(w, rows=128):
    return jnp.zeros((rows, D), jnp.float32).at[: w.shape[0]].set(w)


def kernel(f1_lookup, f2_lookup, f3_content, f3_lookup, f4_content, f4_lookup,
           f5_lookup, f6_time, f6_lookup, f6_week, f6_day,
           emb1, emb2, W3, b3, W4, b4, emb5, W6, b6, week_tab, day_tab):
    i1 = f1_lookup.reshape(B).astype(jnp.int32)
    i2 = f2_lookup.reshape(B).astype(jnp.int32)
    i5 = f5_lookup.reshape(B).astype(jnp.int32)

    g1, g2, g5 = _make_sc_gather()(i1, i2, i5, emb1, emb2, emb5)

    wc = _fourier_w(N_BANDS, 3)
    wt = _fourier_w(N_BANDS, 0)
    wcp = jnp.zeros((1, 128), jnp.float32).at[0, :12].set(wc)
    wtp = jnp.zeros((1, 128), jnp.float32).at[0, :9].set(wt)
    w3s, w3c = _pad_rows(W3[:12]), _pad_rows(W3[12:])
    w4s, w4c = _pad_rows(W4[:12]), _pad_rows(W4[12:])
    w6s, w6c = _pad_rows(W6[:9]), _pad_rows(W6[9:])
    cdt = jnp.zeros((128, D), jnp.float32)
    cdt = cdt.at[:57].set(week_tab).at[64:75].set(day_tab)

    col = lambda a, dt: a.reshape(B, 1).astype(dt)
    vec_spec = pl.BlockSpec((_T, 1), lambda i: (i, 0))
    row_spec = pl.BlockSpec((_T, D), lambda i: (i, 0))
    w_spec = lambda r: pl.BlockSpec((r, 128), lambda i: (0, 0))

    out = pl.pallas_call(
        _tc_body,
        grid=(_GRID,),
        in_specs=[vec_spec] * 8 + [row_spec] * 3
        + [w_spec(1)] * 2 + [w_spec(128)] * 6 + [w_spec(1)] * 3 + [w_spec(128)],
        out_specs=pl.BlockSpec((_T, 6, D), lambda i: (i, 0, 0)),
        out_shape=jax.ShapeDtypeStruct((B, 6, D), jnp.float32),
    )(
        col(f3_content, jnp.float32), col(f3_lookup, jnp.float32),
        col(f4_content, jnp.float32), col(f4_lookup, jnp.float32),
        col(f6_time, jnp.float32), col(f6_lookup, jnp.float32),
        col(f6_week, jnp.int32), col(f6_day, jnp.int32),
        g1, g2, g5,
        wcp, wtp, w3s, w3c, w4s, w4c, w6s, w6c,
        b3.reshape(1, D), b4.reshape(1, D), b6.reshape(1, D), cdt,
    )
    return out.reshape(N, L, 6, D)


# speedup vs baseline: 2.1901x; 1.3316x over previous
import functools
import math

import jax
import jax.numpy as jnp
from jax import lax
from jax.experimental import pallas as pl
from jax.experimental.pallas import tpu as pltpu
from jax.experimental.pallas import tpu_sc as plsc

N, L = 1024, 50
B = N * L
D = 128
N_BANDS = 8

_NC, _NS = 2, 16
_NW = _NC * _NS
_BPW = B // _NW
_HALF = 800
_CH = 80
_NFIRE = _HALF // _CH


def _fourier_w(n_bands, offset):
    steps = n_bands + offset + 1
    w = 2.0 ** jnp.linspace(-float(n_bands), float(offset), steps)
    return (w * math.pi).astype(jnp.float32)


def _make_sc_gather():
    mesh = plsc.VectorSubcoreMesh(core_axis_name="c", subcore_axis_name="s")

    @functools.partial(
        pl.kernel,
        mesh=mesh,
        out_type=(
            jax.ShapeDtypeStruct((B, D), jnp.float32),
            jax.ShapeDtypeStruct((B, D), jnp.float32),
            jax.ShapeDtypeStruct((B, D), jnp.float32),
        ),
        scratch_types=[
            pltpu.VMEM((_HALF,), jnp.int32),
            pltpu.VMEM((_HALF, D), jnp.float32),
            pltpu.SemaphoreType.DMA,
        ],
    )
    def sc_gather(i1, i2, i5, t1, t2, t5, o1, o2, o5, idx_v, rows_v, sem):
        wid = lax.axis_index("s") * _NC + lax.axis_index("c")
        base0 = wid * _BPW

        for ih, th, oh in ((i1, t1, o1), (i2, t2, o2), (i5, t5, o5)):
            def round_body(r, carry, ih=ih, th=th, oh=oh):
                base = base0 + r * _HALF
                pltpu.sync_copy(ih.at[pl.ds(base, _HALF)], idx_v)
                for j in range(_NFIRE):
                    pltpu.async_copy(
                        th.at[idx_v.at[pl.ds(j * _CH, _CH)]],
                        rows_v.at[pl.ds(j * _CH, _CH)],
                        sem,
                    )
                for j in range(_NFIRE):
                    pltpu.make_async_copy(
                        th.at[idx_v.at[pl.ds(j * _CH, _CH)]],
                        rows_v.at[pl.ds(j * _CH, _CH)],
                        sem,
                    ).wait()
                pltpu.sync_copy(rows_v, oh.at[pl.ds(base, _HALF)])
                return carry

            lax.fori_loop(0, _BPW // _HALF, round_body, 0)

    return sc_gather


_T = 1024
_GRID = B // _T


def _tc_body(x3, l3, x4, l4, x6, l6, wk, dy, g1, g2, g5,
             arow, brow, crow, srow, w3cat, w4cat, w6cat, b3, b4, b6, cdt,
             out_ref):
    f32 = jnp.float32

    z3 = x3[...] - l3[...]
    z4 = x4[...] - l4[...]
    z6 = x6[...] - l6[...]
    a = z3 * arow[...] + z4 * brow[...] + z6 * crow[...] + srow[...]
    f = jnp.sin(a)
    e3 = jnp.dot(f, w3cat[...], preferred_element_type=f32) + b3[...]
    e4 = jnp.dot(f, w4cat[...], preferred_element_type=f32) + b4[...]
    e6 = jnp.dot(f, w6cat[...], preferred_element_type=f32) + b6[...]

    lanes = lax.broadcasted_iota(jnp.int32, (_T, 128), 1)
    oh = (lanes == wk[...]).astype(f32) + (lanes == dy[...] + 64).astype(f32)
    e6 += jnp.dot(oh, cdt[...], preferred_element_type=f32)

    out_ref[:, 0, :] = g1[...]
    out_ref[:, 1, :] = g2[...]
    out_ref[:, 2, :] = e3
    out_ref[:, 3, :] = e4
    out_ref[:, 4, :] = g5[...]
    out_ref[:, 5, :] = e6


def _pad_rows(w, rows=128):
    return jnp.zeros((rows, D), jnp.float32).at[: w.shape[0]].set(w)


def kernel(f1_lookup, f2_lookup, f3_content, f3_lookup, f4_content, f4_lookup,
           f5_lookup, f6_time, f6_lookup, f6_week, f6_day,
           emb1, emb2, W3, b3, W4, b4, emb5, W6, b6, week_tab, day_tab):
    i1 = f1_lookup.reshape(B).astype(jnp.int32)
    i2 = f2_lookup.reshape(B).astype(jnp.int32)
    i5 = f5_lookup.reshape(B).astype(jnp.int32)

    g1, g2, g5 = _make_sc_gather()(i1, i2, i5, emb1, emb2, emb5)

    wc = _fourier_w(N_BANDS, 3)
    wt = _fourier_w(N_BANDS, 0)
    hp = math.pi / 2.0
    arow = jnp.zeros((1, 128), jnp.float32).at[0, 0:12].set(wc).at[0, 33:45].set(wc)
    brow = jnp.zeros((1, 128), jnp.float32).at[0, 12:24].set(wc).at[0, 45:57].set(wc)
    crow = jnp.zeros((1, 128), jnp.float32).at[0, 24:33].set(wt).at[0, 57:66].set(wt)
    srow = jnp.zeros((1, 128), jnp.float32).at[0, 33:66].set(hp)
    zw = jnp.zeros((128, D), jnp.float32)
    w3cat = zw.at[0:12].set(W3[:12]).at[33:45].set(W3[12:])
    w4cat = zw.at[12:24].set(W4[:12]).at[45:57].set(W4[12:])
    w6cat = zw.at[24:33].set(W6[:9]).at[57:66].set(W6[9:])
    cdt = zw.at[:57].set(week_tab).at[64:75].set(day_tab)

    col = lambda a, dt: a.reshape(B, 1).astype(dt)
    vec_spec = pl.BlockSpec((_T, 1), lambda i: (i, 0))
    row_spec = pl.BlockSpec((_T, D), lambda i: (i, 0))
    w_spec = lambda r: pl.BlockSpec((r, 128), lambda i: (0, 0))

    out = pl.pallas_call(
        _tc_body,
        grid=(_GRID,),
        in_specs=[vec_spec] * 8 + [row_spec] * 3
        + [w_spec(1)] * 4 + [w_spec(128)] * 3 + [w_spec(1)] * 3 + [w_spec(128)],
        out_specs=pl.BlockSpec((_T, 6, D), lambda i: (i, 0, 0)),
        out_shape=jax.ShapeDtypeStruct((B, 6, D), jnp.float32),
    )(
        col(f3_content, jnp.float32), col(f3_lookup, jnp.float32),
        col(f4_content, jnp.float32), col(f4_lookup, jnp.float32),
        col(f6_time, jnp.float32), col(f6_lookup, jnp.float32),
        col(f6_week, jnp.int32), col(f6_day, jnp.int32),
        g1, g2, g5,
        arow, brow, crow, srow, w3cat, w4cat, w6cat,
        b3.reshape(1, D), b4.reshape(1, D), b6.reshape(1, D), cdt,
    )
    return out.reshape(N, L, 6, D)


# speedup vs baseline: 3.3643x; 1.5362x over previous
import functools
import math

import jax
import jax.numpy as jnp
from jax import lax
from jax.experimental import pallas as pl
from jax.experimental.pallas import tpu as pltpu
from jax.experimental.pallas import tpu_sc as plsc

N, L = 1024, 50
B = N * L
D = 128
N_BANDS = 8

_NC, _NS = 2, 16
_NW = _NC * _NS
_BPW = B // _NW
_HALF = 800
_CH = 80
_NFIRE = _HALF // _CH


def _fourier_w(n_bands, offset):
    steps = n_bands + offset + 1
    w = 2.0 ** jnp.linspace(-float(n_bands), float(offset), steps)
    return (w * math.pi).astype(jnp.float32)


def _make_sc_gather():
    mesh = plsc.VectorSubcoreMesh(core_axis_name="c", subcore_axis_name="s")

    @functools.partial(
        pl.kernel,
        mesh=mesh,
        out_type=(
            jax.ShapeDtypeStruct((B, D), jnp.float32),
            jax.ShapeDtypeStruct((B, D), jnp.float32),
            jax.ShapeDtypeStruct((B, D), jnp.float32),
        ),
        scratch_types=[
            pltpu.VMEM((_HALF,), jnp.int32),
            pltpu.VMEM((_HALF, D), jnp.float32),
            pltpu.SemaphoreType.DMA,
        ],
        compiler_params=pltpu.CompilerParams(use_tc_tiling_on_sc=True),
    )
    def sc_gather(i1, i2, i5, t1, t2, t5, o1, o2, o5, idx_v, rows_v, sem):
        wid = lax.axis_index("s") * _NC + lax.axis_index("c")
        base0 = wid * _BPW

        for ih, th, oh in ((i1, t1, o1), (i2, t2, o2), (i5, t5, o5)):
            def round_body(r, carry, ih=ih, th=th, oh=oh):
                base = base0 + r * _HALF
                pltpu.sync_copy(ih.at[pl.ds(base, _HALF)], idx_v)
                for j in range(_NFIRE):
                    pltpu.async_copy(
                        th.at[idx_v.at[pl.ds(j * _CH, _CH)]],
                        rows_v.at[pl.ds(j * _CH, _CH)],
                        sem,
                    )
                for j in range(_NFIRE):
                    pltpu.make_async_copy(
                        th.at[idx_v.at[pl.ds(j * _CH, _CH)]],
                        rows_v.at[pl.ds(j * _CH, _CH)],
                        sem,
                    ).wait()
                pltpu.sync_copy(rows_v, oh.at[pl.ds(base, _HALF)])
                return carry

            lax.fori_loop(0, _BPW // _HALF, round_body, 0)

    return sc_gather


_T = 1024
_GRID = B // _T


def _tc_body(x3, l3, x4, l4, x6, l6, wk, dy, g1, g2, g5,
             arow, brow, crow, srow, w3cat, w4cat, w6cat, b3, b4, b6, cdt,
             out_ref):
    f32 = jnp.float32

    z3 = x3[...] - l3[...]
    z4 = x4[...] - l4[...]
    z6 = x6[...] - l6[...]
    a = z3 * arow[...] + z4 * brow[...] + z6 * crow[...] + srow[...]
    f = jnp.sin(a)
    e3 = jnp.dot(f, w3cat[...], preferred_element_type=f32) + b3[...]
    e4 = jnp.dot(f, w4cat[...], preferred_element_type=f32) + b4[...]
    e6 = jnp.dot(f, w6cat[...], preferred_element_type=f32) + b6[...]

    lanes = lax.broadcasted_iota(jnp.int32, (_T, 128), 1)
    oh = (lanes == wk[...]).astype(f32) + (lanes == dy[...] + 64).astype(f32)
    e6 += jnp.dot(oh, cdt[...], preferred_element_type=f32)

    out_ref[0, 0, :, :] = g1[...]
    out_ref[0, 1, :, :] = g2[...]
    out_ref[0, 2, :, :] = e3
    out_ref[0, 3, :, :] = e4
    out_ref[0, 4, :, :] = g5[...]
    out_ref[0, 5, :, :] = e6


def _pad_rows(w, rows=128):
    return jnp.zeros((rows, D), jnp.float32).at[: w.shape[0]].set(w)


def kernel(f1_lookup, f2_lookup, f3_content, f3_lookup, f4_content, f4_lookup,
           f5_lookup, f6_time, f6_lookup, f6_week, f6_day,
           emb1, emb2, W3, b3, W4, b4, emb5, W6, b6, week_tab, day_tab):
    i1 = f1_lookup.T.reshape(B).astype(jnp.int32)
    i2 = f2_lookup.T.reshape(B).astype(jnp.int32)
    i5 = f5_lookup.T.reshape(B).astype(jnp.int32)

    g1, g2, g5 = _make_sc_gather()(i1, i2, i5, emb1, emb2, emb5)

    wc = _fourier_w(N_BANDS, 3)
    wt = _fourier_w(N_BANDS, 0)
    hp = math.pi / 2.0
    arow = jnp.zeros((1, 128), jnp.float32).at[0, 0:12].set(wc).at[0, 33:45].set(wc)
    brow = jnp.zeros((1, 128), jnp.float32).at[0, 12:24].set(wc).at[0, 45:57].set(wc)
    crow = jnp.zeros((1, 128), jnp.float32).at[0, 24:33].set(wt).at[0, 57:66].set(wt)
    srow = jnp.zeros((1, 128), jnp.float32).at[0, 33:66].set(hp)
    zw = jnp.zeros((128, D), jnp.float32)
    w3cat = zw.at[0:12].set(W3[:12]).at[33:45].set(W3[12:])
    w4cat = zw.at[12:24].set(W4[:12]).at[45:57].set(W4[12:])
    w6cat = zw.at[24:33].set(W6[:9]).at[57:66].set(W6[9:])
    cdt = zw.at[:57].set(week_tab).at[64:75].set(day_tab)

    col = lambda a, dt: a.T.reshape(B, 1).astype(dt)
    vec_spec = pl.BlockSpec((_T, 1), lambda i: (i, 0))
    row_spec = pl.BlockSpec((_T, D), lambda i: (i, 0))
    w_spec = lambda r: pl.BlockSpec((r, 128), lambda i: (0, 0))

    out = pl.pallas_call(
        _tc_body,
        grid=(_GRID,),
        in_specs=[vec_spec] * 8 + [row_spec] * 3
        + [w_spec(1)] * 4 + [w_spec(128)] * 3 + [w_spec(1)] * 3 + [w_spec(128)],
        out_specs=pl.BlockSpec((1, 6, _T, D), lambda i: (i, 0, 0, 0)),
        out_shape=jax.ShapeDtypeStruct((L, 6, N, D), jnp.float32),
    )(
        col(f3_content, jnp.float32), col(f3_lookup, jnp.float32),
        col(f4_content, jnp.float32), col(f4_lookup, jnp.float32),
        col(f6_time, jnp.float32), col(f6_lookup, jnp.float32),
        col(f6_week, jnp.int32), col(f6_day, jnp.int32),
        g1, g2, g5,
        arow, brow, crow, srow, w3cat, w4cat, w6cat,
        b3.reshape(1, D), b4.reshape(1, D), b6.reshape(1, D), cdt,
    )
    return jnp.transpose(out, (2, 0, 1, 3))
